# baseline (speedup 1.0000x reference)
import functools
import math

import jax
import jax.numpy as jnp
from jax import lax
from jax.experimental import pallas as pl
from jax.experimental.pallas import tpu as pltpu
from jax.experimental.pallas import tpu_sc as plsc

N_HID = 62
MAX_LEN = 3000
N_ROWS = 4096
D_PAD = 128

_W_HMS = (3600.0 / 3661.0, 60.0 / 3661.0, 1.0 / 3661.0)

_NC = 2
_NS = 16
_NW = _NC * _NS
_RPW = N_ROWS // _NW


def _tc_table_body(table_ref, w_ref, b_ref, out_ref):
    prod = lax.dot_general(
        table_ref[...], w_ref[...],
        (((1,), (1,)), ((), ())),
        preferred_element_type=jnp.float32,
    )
    h = prod + b_ref[...]
    out_ref[...] = jnp.concatenate(
        [h, jnp.zeros((MAX_LEN, D_PAD - N_HID), jnp.float32)], axis=1)


_tc_table = pl.pallas_call(
    _tc_table_body,
    out_shape=jax.ShapeDtypeStruct((MAX_LEN, D_PAD), jnp.float32),
)


def _sc_body(t_hbm, tw_hbm, out_hbm, tv, rows, acc, sem):
    wid = lax.axis_index("s") * _NC + lax.axis_index("c")
    base = wid * _RPW

    pltpu.sync_copy(t_hbm.at[pl.ds(base * 3, 3 * _RPW)], tv)

    cps = [
        pltpu.async_copy(tw_hbm.at[tv.at[pl.ds(g * _RPW, _RPW)]],
                         rows.at[pl.ds(g * _RPW, _RPW)], sem)
        for g in range(3)
    ]
    for cp in cps:
        cp.wait()

    def body(i2, carry):
        for u in range(4):
            i = i2 * 4 + u
            for c in range(4):
                s = pl.ds(c * 16, 16)
                acc[i, s] = (_W_HMS[0] * rows[3 * i, s]
                             + _W_HMS[1] * rows[3 * i + 1, s]
                             + _W_HMS[2] * rows[3 * i + 2, s])
        return carry

    lax.fori_loop(0, _RPW // 4, body, 0)

    pltpu.sync_copy(acc, out_hbm.at[pl.ds(base, _RPW)])


@functools.cache
def _sc_gather():
    return pl.kernel(
        _sc_body,
        out_type=jax.ShapeDtypeStruct((N_ROWS, D_PAD), jnp.float32),
        mesh=plsc.VectorSubcoreMesh(core_axis_name="c", subcore_axis_name="s"),
        scratch_types=[
            pltpu.VMEM((3 * _RPW,), jnp.int32),
            pltpu.VMEM((3 * _RPW, D_PAD), jnp.float32),
            pltpu.VMEM((_RPW, D_PAD), jnp.float32),
            pltpu.SemaphoreType.DMA,
        ],
    )


def kernel(t, table, W, b):
    tw = _tc_table(table, W, b.reshape(1, N_HID))
    out = _sc_gather()(t.reshape(-1), tw)
    return out[:, :N_HID]


# speedup vs baseline: 1.0001x; 1.0001x over previous
import functools
import math

import jax
import jax.numpy as jnp
from jax import lax
from jax.experimental import pallas as pl
from jax.experimental.pallas import tpu as pltpu
from jax.experimental.pallas import tpu_sc as plsc

N_HID = 62
MAX_LEN = 3000
N_ROWS = 4096
D_PAD = 128

_W_HMS = (3600.0 / 3661.0, 60.0 / 3661.0, 1.0 / 3661.0)

_NC = 2
_NS = 16
_NW = _NC * _NS
_RPW = N_ROWS // _NW


def _tc_table_body(table_ref, w_ref, b_ref, out_ref):
    prod = lax.dot_general(
        table_ref[...], w_ref[...],
        (((1,), (1,)), ((), ())),
        preferred_element_type=jnp.float32,
    )
    h = prod + b_ref[...]
    out_ref[...] = jnp.concatenate(
        [h, jnp.zeros((MAX_LEN, D_PAD - N_HID), jnp.float32)], axis=1)


_tc_table = pl.pallas_call(
    _tc_table_body,
    out_shape=jax.ShapeDtypeStruct((MAX_LEN, D_PAD), jnp.float32),
)


def _sc_body(t_hbm, tw_hbm, out_hbm, tv, rows, acc, sem):
    wid = lax.axis_index("s") * _NC + lax.axis_index("c")
    base = wid * _RPW

    pltpu.sync_copy(t_hbm.at[pl.ds(base * 3, 3 * _RPW)], tv)

    cps = [
        pltpu.async_copy(tw_hbm.at[tv.at[pl.ds(g * _RPW, _RPW)]],
                         rows.at[pl.ds(g * _RPW, _RPW)], sem)
        for g in range(3)
    ]
    for cp in cps:
        cp.wait()

    def body(i2, carry):
        for u in range(4):
            i = i2 * 4 + u
            for off in (0, 16, 32, N_HID - 16):
                s = pl.ds(off, 16)
                acc[i, s] = (_W_HMS[0] * rows[3 * i, s]
                             + _W_HMS[1] * rows[3 * i + 1, s]
                             + _W_HMS[2] * rows[3 * i + 2, s])
        return carry

    lax.fori_loop(0, _RPW // 4, body, 0)

    pltpu.sync_copy(acc, out_hbm.at[pl.ds(base, _RPW)])


@functools.cache
def _sc_gather():
    return pl.kernel(
        _sc_body,
        out_type=jax.ShapeDtypeStruct((N_ROWS, N_HID), jnp.float32),
        mesh=plsc.VectorSubcoreMesh(core_axis_name="c", subcore_axis_name="s"),
        scratch_types=[
            pltpu.VMEM((3 * _RPW,), jnp.int32),
            pltpu.VMEM((3 * _RPW, D_PAD), jnp.float32),
            pltpu.VMEM((_RPW, N_HID), jnp.float32),
            pltpu.SemaphoreType.DMA,
        ],
    )


def kernel(t, table, W, b):
    tw = _tc_table(table, W, b.reshape(1, N_HID))
    return _sc_gather()(t.reshape(-1), tw)
